# Initial kernel scaffold; baseline (speedup 1.0000x reference)
import jax
import jax.numpy as jnp
from jax.experimental import pallas as pl


def kernel(x, w1, b1, w2, b2):
    raise NotImplementedError("write your pallas kernel here")


# speedup vs baseline: 3.1020x; 3.1020x over previous
import functools

import jax
import jax.numpy as jnp
from jax.experimental import pallas as pl
from jax.experimental.pallas import tpu as pltpu


def _se_fused_kernel(x_ref, w1t_ref, b1_ref, w2t_ref, b2_ref, o_ref, *, inv_s):
    x = x_ref[0]
    s = jnp.sum(x, axis=1, keepdims=True) * inv_s
    h = jnp.maximum(
        jnp.dot(w1t_ref[...], s, preferred_element_type=jnp.float32)
        + b1_ref[...], 0.0)
    g = jax.nn.sigmoid(
        jnp.dot(w2t_ref[...], h, preferred_element_type=jnp.float32)
        + b2_ref[...])
    o_ref[0] = x * g


def kernel(x, w1, b1, w2, b2):
    B, C, D, H, W = x.shape
    S = D * H * W
    Cr = w1.shape[1]

    x3 = x.reshape(B, C, S)
    w1t = w1.T
    w2t = w2.T
    b1c = b1.reshape(Cr, 1)
    b2c = b2.reshape(C, 1)

    out = pl.pallas_call(
        functools.partial(_se_fused_kernel, inv_s=float(1.0 / S)),
        out_shape=jax.ShapeDtypeStruct((B, C, S), x.dtype),
        grid=(B,),
        in_specs=[
            pl.BlockSpec((1, C, S), lambda b: (b, 0, 0)),
            pl.BlockSpec((Cr, C), lambda b: (0, 0)),
            pl.BlockSpec((Cr, 1), lambda b: (0, 0)),
            pl.BlockSpec((C, Cr), lambda b: (0, 0)),
            pl.BlockSpec((C, 1), lambda b: (0, 0)),
        ],
        out_specs=pl.BlockSpec((1, C, S), lambda b: (b, 0, 0)),
        compiler_params=pltpu.CompilerParams(
            dimension_semantics=("parallel",),
            vmem_limit_bytes=48 * 1024 * 1024),
    )(x3, w1t, b1c, w2t, b2c)

    return out.reshape(B, C, D, H, W)
